# Initial kernel scaffold; baseline (speedup 1.0000x reference)
import jax
import jax.numpy as jnp
from jax.experimental import pallas as pl


def kernel(nodes, to_neighs, u2e, gate_W, gate_b, att1_W, att1_b, att2_W, att2_b, att3_W, att3_b):
    raise NotImplementedError("write your pallas kernel here")


# speedup vs baseline: 3.0380x; 3.0380x over previous
import functools

import jax
import jax.numpy as jnp
from jax import lax
from jax.experimental import pallas as pl
from jax.experimental.pallas import tpu as pltpu
from jax.experimental.pallas import tpu_sc as plsc

D = 128
K = 32

_NC = 2
_NS = 16
_NW = _NC * _NS

_CH = 400


def _sc_gather(u2e, nidx, sidx):
    RN = nidx.shape[0]
    RS = sidx.shape[0]
    n_per_w = RN // _NW
    s_per_w = RS // _NW
    n_chunks = n_per_w // _CH
    mesh = plsc.VectorSubcoreMesh(core_axis_name="c", subcore_axis_name="s")

    @functools.partial(
        pl.kernel,
        mesh=mesh,
        out_type=[
            jax.ShapeDtypeStruct((RN, D), jnp.float32),
            jax.ShapeDtypeStruct((RS, D), jnp.float32),
        ],
        scratch_types=[
            pltpu.VMEM((_CH,), jnp.int32),
            pltpu.VMEM((_CH, D), jnp.float32),
            pltpu.VMEM((s_per_w,), jnp.int32),
            pltpu.VMEM((s_per_w, D), jnp.float32),
            pltpu.SemaphoreType.DMA,
        ],
    )
    def gather_k(u2e_hbm, nidx_hbm, sidx_hbm, out_n_hbm, out_s_hbm,
                 idx_v, rows_v, idx_s, rows_s, sem):
        wid = lax.axis_index("s") * _NC + lax.axis_index("c")
        base_n = wid * n_per_w
        base_s = wid * s_per_w

        def body(i, carry):
            off = base_n + i * _CH
            pltpu.sync_copy(nidx_hbm.at[pl.ds(off, _CH)], idx_v)
            pltpu.async_copy(u2e_hbm.at[idx_v], rows_v, sem).wait()
            pltpu.sync_copy(rows_v, out_n_hbm.at[pl.ds(off, _CH)])
            return carry

        lax.fori_loop(0, n_chunks, body, 0)

        pltpu.sync_copy(sidx_hbm.at[pl.ds(base_s, s_per_w)], idx_s)
        pltpu.async_copy(u2e_hbm.at[idx_s], rows_s, sem).wait()
        pltpu.sync_copy(rows_s, out_s_hbm.at[pl.ds(base_s, s_per_w)])

    return gather_k(u2e, nidx, sidx)


def _dense_body(eu_ref, ur_ref, gWt_ref, gWb_ref, gb_ref,
                a1t_ref, a1b_w_ref, a1b_ref, a2W_ref, a2b_ref, a3w_ref,
                out_ref):
    B = ur_ref.shape[0]
    eu = eu_ref[...]
    ur = ur_ref[...]

    sg = jnp.dot(ur, gWb_ref[...], preferred_element_type=jnp.float32) + gb_ref[...]
    sa = jnp.dot(ur, a1b_w_ref[...], preferred_element_type=jnp.float32) + a1b_ref[...]

    zg = jnp.dot(eu, gWt_ref[...], preferred_element_type=jnp.float32)
    eu3 = eu.reshape(B, K, D)
    g = jax.nn.sigmoid(zg.reshape(B, K, D) + sg[:, None, :])
    e_g3 = g * eu3 + (1.0 - g) * ur[:, None, :]

    x1 = jnp.dot(e_g3.reshape(B * K, D), a1t_ref[...],
                 preferred_element_type=jnp.float32)
    x1 = jnp.maximum(x1.reshape(B, K, D) + sa[:, None, :], 0.0).reshape(B * K, D)
    x2 = jnp.maximum(
        jnp.dot(x1, a2W_ref[...], preferred_element_type=jnp.float32) + a2b_ref[...],
        0.0)

    s = jnp.sum(x2.reshape(B, K, D) * a3w_ref[...][None, :, :], axis=-1)
    m = jnp.max(s, axis=1, keepdims=True)
    w = jnp.exp(s - m)
    att = w / jnp.sum(w, axis=1, keepdims=True)

    out_ref[...] = jnp.sum(e_g3 * att[:, :, None], axis=1)


def _dense_call(eu, ur, gWt, gWb, gb, a1t, a1b_w, a1b, a2W, a2b, a3w, n_nodes, B):
    grid = n_nodes // B
    full = lambda i: (0, 0)
    return pl.pallas_call(
        _dense_body,
        grid=(grid,),
        in_specs=[
            pl.BlockSpec((B * K, D), lambda i: (i, 0)),
            pl.BlockSpec((B, D), lambda i: (i, 0)),
            pl.BlockSpec((D, D), full),
            pl.BlockSpec((D, D), full),
            pl.BlockSpec((1, D), full),
            pl.BlockSpec((D, D), full),
            pl.BlockSpec((D, D), full),
            pl.BlockSpec((1, D), full),
            pl.BlockSpec((D, D), full),
            pl.BlockSpec((1, D), full),
            pl.BlockSpec((1, D), full),
        ],
        out_specs=pl.BlockSpec((B, D), lambda i: (i, 0)),
        out_shape=jax.ShapeDtypeStruct((n_nodes, D), jnp.float32),
    )(eu, ur, gWt, gWb, gb, a1t, a1b_w, a1b, a2W, a2b, a3w)


def kernel(nodes, to_neighs, u2e, gate_W, gate_b, att1_W, att1_b,
           att2_W, att2_b, att3_W, att3_b):
    N = to_neighs.shape[0]

    nidx = to_neighs.reshape(-1).astype(jnp.int32)
    s_pad = (-N) % (8 * _NW)
    sidx = jnp.pad(nodes.astype(jnp.int32), (0, s_pad))

    eu, ur = _sc_gather(u2e, nidx, sidx)

    gWt, gWb = gate_W[:D], gate_W[D:]
    a1t, a1b_w = att1_W[:D], att1_W[D:]
    gb = gate_b.reshape(1, D)
    a1b = att1_b.reshape(1, D)
    a2b = att2_b.reshape(1, D)
    a3w = att3_W.reshape(1, D)

    return _dense_call(eu, ur, gWt, gWb, gb, a1t, a1b_w, a1b, a2W=att2_W,
                       a2b=a2b, a3w=a3w, n_nodes=N, B=200)


# speedup vs baseline: 3.1383x; 1.0330x over previous
import functools

import jax
import jax.numpy as jnp
from jax import lax
from jax.experimental import pallas as pl
from jax.experimental.pallas import tpu as pltpu
from jax.experimental.pallas import tpu_sc as plsc

D = 128
K = 32

_NC = 2
_NS = 16
_NW = _NC * _NS

_CH = 344


def _sc_gather(u2e, idx):
    R = idx.shape[0]
    r_per_w = R // _NW
    n_chunks = r_per_w // _CH
    mesh = plsc.VectorSubcoreMesh(core_axis_name="c", subcore_axis_name="s")

    @functools.partial(
        pl.kernel,
        mesh=mesh,
        out_type=jax.ShapeDtypeStruct((R, D), jnp.float32),
        scratch_types=[
            pltpu.VMEM((r_per_w,), jnp.int32),
            pltpu.VMEM((_CH, D), jnp.float32),
            pltpu.VMEM((_CH, D), jnp.float32),
            pltpu.SemaphoreType.DMA,
            pltpu.SemaphoreType.DMA,
            pltpu.SemaphoreType.DMA,
            pltpu.SemaphoreType.DMA,
        ],
    )
    def gather_k(u2e_hbm, idx_hbm, out_hbm, idx_all, rows0, rows1,
                 sg0, sg1, so0, so1):
        wid = lax.axis_index("s") * _NC + lax.axis_index("c")
        base = wid * r_per_w
        pltpu.sync_copy(idx_hbm.at[pl.ds(base, r_per_w)], idx_all)

        rows = (rows0, rows1)
        sg = (sg0, sg1)
        so = (so0, so1)

        def g_copy(i, b):
            return pltpu.make_async_copy(
                u2e_hbm.at[idx_all.at[pl.ds(i * _CH, _CH)]], rows[b], sg[b])

        def o_copy(i, b):
            return pltpu.make_async_copy(
                rows[b], out_hbm.at[pl.ds(base + i * _CH, _CH)], so[b])

        g_copy(0, 0).start()
        g_copy(1, 1).start()

        def body(p, carry):
            for b in range(2):
                i = 2 * p + b
                g_copy(i, b).wait()
                o_copy(i, b).start()
                o_copy(i, b).wait()
                g_copy(i + 2, b).start()
            return carry

        lax.fori_loop(0, n_chunks // 2 - 1, body, 0)

        for b in range(2):
            i = n_chunks - 2 + b
            g_copy(i, b).wait()
            o_copy(i, b).start()
            o_copy(i, b).wait()

    return gather_k(u2e, idx)


def _dense_body(eu_ref, ur_ref, gWt_ref, gWb_ref, gb_ref,
                a1t_ref, a1b_w_ref, a1b_ref, a2W_ref, a2b_ref, a3w_ref,
                out_ref):
    B = ur_ref.shape[0]
    eu = eu_ref[...]
    ur = ur_ref[...]

    sg = jnp.dot(ur, gWb_ref[...], preferred_element_type=jnp.float32) + gb_ref[...]
    sa = jnp.dot(ur, a1b_w_ref[...], preferred_element_type=jnp.float32) + a1b_ref[...]

    zg = jnp.dot(eu, gWt_ref[...], preferred_element_type=jnp.float32)
    eu3 = eu.reshape(B, K, D)
    g = jax.nn.sigmoid(zg.reshape(B, K, D) + sg[:, None, :])
    e_g3 = g * eu3 + (1.0 - g) * ur[:, None, :]

    x1 = jnp.dot(e_g3.reshape(B * K, D), a1t_ref[...],
                 preferred_element_type=jnp.float32)
    x1 = jnp.maximum(x1.reshape(B, K, D) + sa[:, None, :], 0.0).reshape(B * K, D)
    x2 = jnp.maximum(
        jnp.dot(x1, a2W_ref[...], preferred_element_type=jnp.float32) + a2b_ref[...],
        0.0)

    s = jnp.sum(x2.reshape(B, K, D) * a3w_ref[...][None, :, :], axis=-1)
    m = jnp.max(s, axis=1, keepdims=True)
    w = jnp.exp(s - m)
    att = w / jnp.sum(w, axis=1, keepdims=True)

    out_ref[...] = jnp.sum(e_g3 * att[:, :, None], axis=1)


def _dense_call(rows, gWt, gWb, gb, a1t, a1b_w, a1b, a2W, a2b, a3w,
                n_nodes, B, self_row0):
    grid = n_nodes // B
    full = lambda i: (0, 0)
    self_blk0 = self_row0 // B
    return pl.pallas_call(
        _dense_body,
        grid=(grid,),
        in_specs=[
            pl.BlockSpec((B * K, D), lambda i: (i, 0)),
            pl.BlockSpec((B, D), lambda i: (self_blk0 + i, 0)),
            pl.BlockSpec((D, D), full),
            pl.BlockSpec((D, D), full),
            pl.BlockSpec((1, D), full),
            pl.BlockSpec((D, D), full),
            pl.BlockSpec((D, D), full),
            pl.BlockSpec((1, D), full),
            pl.BlockSpec((D, D), full),
            pl.BlockSpec((1, D), full),
            pl.BlockSpec((1, D), full),
        ],
        out_specs=pl.BlockSpec((B, D), lambda i: (i, 0)),
        out_shape=jax.ShapeDtypeStruct((n_nodes, D), jnp.float32),
    )(rows, rows, gWt, gWb, gb, a1t, a1b_w, a1b, a2W, a2b, a3w)


def kernel(nodes, to_neighs, u2e, gate_W, gate_b, att1_W, att1_b,
           att2_W, att2_b, att3_W, att3_b):
    N = to_neighs.shape[0]
    n_rows = N * K
    total = n_rows + N
    pad = (-total) % (_NW * _CH)
    idx = jnp.concatenate([
        to_neighs.reshape(-1).astype(jnp.int32),
        nodes.astype(jnp.int32),
        jnp.zeros((pad,), jnp.int32),
    ])

    rows = _sc_gather(u2e, idx)

    gWt, gWb = gate_W[:D], gate_W[D:]
    a1t, a1b_w = att1_W[:D], att1_W[D:]
    gb = gate_b.reshape(1, D)
    a1b = att1_b.reshape(1, D)
    a2b = att2_b.reshape(1, D)
    a3w = att3_W.reshape(1, D)

    return _dense_call(rows, gWt, gWb, gb, a1t, a1b_w, a1b, att2_W, a2b, a3w,
                       n_nodes=N, B=200, self_row0=n_rows)
